# baseline (speedup 1.0000x reference)
import jax
import jax.numpy as jnp
from jax import lax
from jax.experimental import pallas as pl
from jax.experimental.pallas import tpu as pltpu
from jax.experimental.pallas import tpu_sc as plsc

_NUM_CORES = 2
_NUM_SUBCORES = 16
_NUM_WORKERS = _NUM_CORES * _NUM_SUBCORES

_SEQ = 50
_BATCH = 16384
_D = 32
_LANES = 16
_BB = _BATCH // 128
_BB_PER_W = _BB // _NUM_WORKERS
_ROWS = _BB_PER_W * 128


def _gather_body(table_hbm, idx_hbm, out_hbm, idx_v, gath_v, trans_v, *sems):
    isem = sems[0]
    gsems = sems[1:3]
    wsems = sems[3:5]
    wid = lax.axis_index("s") * _NUM_CORES + lax.axis_index("c")
    b2_base = 128 * _BB_PER_W * wid

    for j in range(_SEQ):
        pltpu.async_copy(
            idx_hbm.at[pl.ds(j * _BATCH + b2_base, _ROWS)],
            idx_v.at[j],
            isem,
        )
    for j in range(_SEQ):
        pltpu.make_async_copy(
            idx_hbm.at[pl.ds(j * _BATCH + b2_base, _ROWS)],
            idx_v.at[j],
            isem,
        ).wait()

    iotas = [
        jnp.arange(m * _LANES, m * _LANES + _LANES, dtype=jnp.int32)
        for m in range(_ROWS // _LANES)
    ]

    def start_gather(j, b):
        pltpu.async_copy(
            table_hbm.at[idx_v.at[j]], gath_v.at[b], gsems[b]
        )

    def wb_descs(j, b):
        return [
            pltpu.make_async_copy(
                trans_v.at[b, fb],
                out_hbm.at[j, fb, pl.ds(_BB_PER_W * wid, _BB_PER_W)],
                wsems[b],
            )
            for fb in range(_D // 8)
        ]

    def transpose(b):
        @pl.loop(0, _D)
        def _(d):
            fb = d // 8
            fi = d % 8
            col = jnp.full((_LANES,), d, dtype=jnp.int32)
            for m in range(_ROWS // _LANES):
                v = plsc.load_gather(gath_v.at[b], [iotas[m], col])
                trans_v[b, fb, m // 8, fi, pl.ds((m % 8) * _LANES, _LANES)] = v

    def step(j, b):
        pltpu.make_async_copy(
            table_hbm.at[idx_v.at[j]], gath_v.at[b], gsems[b]
        ).wait()
        jn = j + 1
        if isinstance(jn, int):
            if jn < _SEQ:
                start_gather(jn, 1 - b)
        else:
            @pl.when(jn < _SEQ)
            def _():
                start_gather(jn, 1 - b)

    start_gather(0, 0)
    for j in (0, 1):
        step(j, j % 2)
        transpose(j % 2)
        for dsc in wb_descs(j, j % 2):
            dsc.start()

    @pl.loop(2, _SEQ, step=2)
    def _(jo):
        for db in range(2):
            j = jo + db
            step(j, db)
            for dsc in wb_descs(j, db):
                dsc.wait()
            transpose(db)
            for dsc in wb_descs(j, db):
                dsc.start()

    for j in (_SEQ - 2, _SEQ - 1):
        for dsc in wb_descs(j, j % 2):
            dsc.wait()


@jax.jit
def _gather(table, idx_t):
    mesh = plsc.VectorSubcoreMesh(
        core_axis_name="c", subcore_axis_name="s",
        num_cores=_NUM_CORES, num_subcores=_NUM_SUBCORES,
    )
    return pl.kernel(
        _gather_body,
        out_type=jax.ShapeDtypeStruct(
            (_SEQ, _D // 8, _BB, 8, 128), jnp.float32
        ),
        mesh=mesh,
        scratch_types=[
            pltpu.VMEM((_SEQ, _ROWS), jnp.int32),
            pltpu.VMEM((2, _ROWS, _D), jnp.float32),
            pltpu.VMEM((2, _D // 8, _BB_PER_W, 8, 128), jnp.float32),
        ] + [pltpu.SemaphoreType.DMA] * 5,
        compiler_params=pltpu.CompilerParams(
            use_tc_tiling_on_sc=False, needs_layout_passes=False
        ),
    )(table, idx_t)


def kernel(inputs, entity_table, relation_table):
    idx_t = jnp.ravel(jnp.swapaxes(inputs, 0, 1)).astype(jnp.int32)
    out5 = _gather(entity_table, idx_t)
    return jnp.transpose(out5, (2, 4, 0, 1, 3)).reshape(_BATCH, _SEQ, _D)


# speedup vs baseline: 1.5811x; 1.5811x over previous
import jax
import jax.numpy as jnp
from jax import lax
from jax.experimental import pallas as pl
from jax.experimental.pallas import tpu as pltpu
from jax.experimental.pallas import tpu_sc as plsc

_NUM_CORES = 2
_NUM_SUBCORES = 16
_NUM_WORKERS = _NUM_CORES * _NUM_SUBCORES

_SEQ = 50
_BATCH = 16384
_D = 32
_LANES = 16
_BB = _BATCH // 128
_BB_PER_W = _BB // _NUM_WORKERS
_ROWS = _BB_PER_W * 128


def _gather_body(table_hbm, idx_hbm, out_hbm, idx_v, gath_v, trans_v, *sems):
    isem = sems[0]
    gsems = sems[1:3]
    wsems = sems[3:5]
    wid = lax.axis_index("s") * _NUM_CORES + lax.axis_index("c")
    b2_base = 128 * _BB_PER_W * wid

    for j in range(_SEQ):
        pltpu.async_copy(
            idx_hbm.at[pl.ds(j * _BATCH + b2_base, _ROWS)],
            idx_v.at[j],
            isem,
        )
    for j in range(_SEQ):
        pltpu.make_async_copy(
            idx_hbm.at[pl.ds(j * _BATCH + b2_base, _ROWS)],
            idx_v.at[j],
            isem,
        ).wait()

    iota = jnp.arange(_LANES, dtype=jnp.int32)
    diag_cols = [
        jax.lax.bitwise_and(iota + k, _LANES - 1) for k in range(_LANES)
    ]

    def start_gather(j, b):
        pltpu.async_copy(
            table_hbm.at[idx_v.at[j]], gath_v.at[b], gsems[b]
        )

    def wb_descs(j, b):
        return [
            pltpu.make_async_copy(
                trans_v.at[b, fb],
                out_hbm.at[j, fb, pl.ds(_BB_PER_W * wid, _BB_PER_W)],
                wsems[b],
            )
            for fb in range(_D // 8)
        ]

    def transpose(b):
        @pl.loop(0, _ROWS // _LANES)
        def _(m):
            rb = m * _LANES
            row_idx = iota + rb
            l_v = iota + rb % 128
            bbl_v = jnp.full((_LANES,), rb // 128, dtype=jnp.int32)
            for cb in (0, _LANES):
                for k in range(_LANES):
                    col = diag_cols[k] + cb if cb else diag_cols[k]
                    v = plsc.load_gather(gath_v.at[b], [row_idx, col])
                    fb_v = jax.lax.shift_right_logical(col, 3)
                    fi_v = jax.lax.bitwise_and(col, 7)
                    plsc.store_scatter(
                        trans_v.at[b], [fb_v, bbl_v, fi_v, l_v], v
                    )

    def step(j, b):
        pltpu.make_async_copy(
            table_hbm.at[idx_v.at[j]], gath_v.at[b], gsems[b]
        ).wait()
        jn = j + 1
        if isinstance(jn, int):
            if jn < _SEQ:
                start_gather(jn, 1 - b)
        else:
            @pl.when(jn < _SEQ)
            def _():
                start_gather(jn, 1 - b)

    start_gather(0, 0)
    for j in (0, 1):
        step(j, j % 2)
        transpose(j % 2)
        for dsc in wb_descs(j, j % 2):
            dsc.start()

    @pl.loop(2, _SEQ, step=2)
    def _(jo):
        for db in range(2):
            j = jo + db
            step(j, db)
            for dsc in wb_descs(j, db):
                dsc.wait()
            transpose(db)
            for dsc in wb_descs(j, db):
                dsc.start()

    for j in (_SEQ - 2, _SEQ - 1):
        for dsc in wb_descs(j, j % 2):
            dsc.wait()


@jax.jit
def _gather(table, idx_t):
    mesh = plsc.VectorSubcoreMesh(
        core_axis_name="c", subcore_axis_name="s",
        num_cores=_NUM_CORES, num_subcores=_NUM_SUBCORES,
    )
    return pl.kernel(
        _gather_body,
        out_type=jax.ShapeDtypeStruct(
            (_SEQ, _D // 8, _BB, 8, 128), jnp.float32
        ),
        mesh=mesh,
        scratch_types=[
            pltpu.VMEM((_SEQ, _ROWS), jnp.int32),
            pltpu.VMEM((2, _ROWS, _D), jnp.float32),
            pltpu.VMEM((2, _D // 8, _BB_PER_W, 8, 128), jnp.float32),
        ] + [pltpu.SemaphoreType.DMA] * 5,
        compiler_params=pltpu.CompilerParams(
            use_tc_tiling_on_sc=False, needs_layout_passes=False
        ),
    )(table, idx_t)


def kernel(inputs, entity_table, relation_table):
    idx_t = jnp.ravel(jnp.swapaxes(inputs, 0, 1)).astype(jnp.int32)
    out5 = _gather(entity_table, idx_t)
    return jnp.transpose(out5, (2, 4, 0, 1, 3)).reshape(_BATCH, _SEQ, _D)
